# baseline (speedup 1.0000x reference)
import functools
import math

import jax
import jax.numpy as jnp
from jax.experimental import pallas as pl
from jax.experimental.pallas import tpu as pltpu

THRESHOLD = 0.05
WEIGHT = 1.0

B = 8
N = 16384
M = 2048
BN = 2048
NBLK = N // BN
PCH = 256
NPCH = M // PCH
RAD = math.sqrt(THRESHOLD) + 1e-4


def _loss_kernel(loc_ref, hic_ref, pg_ref, at_ref, c2l_ref, out_ref,
                 s_ref, n_ref, p2s_ref):
    b = pl.program_id(0)
    i = pl.program_id(1)
    step = b * NBLK + i

    @pl.when(step == 0)
    def _init():
        s_ref[...] = jnp.zeros_like(s_ref)
        n_ref[...] = jnp.zeros_like(n_ref)

    at = at_ref[0]
    c2l = c2l_ref[0]

    @pl.when(i == 0)
    def _batch_setup():
        pg0 = pg_ref[0]
        p2 = jnp.sum(pg0 * pg0, axis=1, keepdims=True)
        p2s_ref[...] = jnp.broadcast_to(p2, (M, 128))

    lo = loc_ref[step]
    hi = hic_ref[step]

    def body(t, acc):
        pgc = pg_ref[0, pl.ds(t * PCH, PCH), :]
        p2c = p2s_ref[pl.ds(t * PCH, PCH), :]
        newacc = []
        for q in range(BN // 128):
            atc = at[:, q * 128:(q + 1) * 128]
            e = jnp.dot(pgc, atc, preferred_element_type=jnp.float32)
            e = e + p2c
            rows = PCH
            while rows > 8:
                half = rows // 2
                e = jnp.minimum(e[:half], e[half:rows])
                rows = half
            newacc.append(jnp.minimum(acc[:, q * 128:(q + 1) * 128], e))
        return jnp.concatenate(newacc, axis=1)

    acc0 = jnp.full((8, BN), jnp.inf, jnp.float32)
    acc = jax.lax.fori_loop(lo, hi, body, acc0)

    dmin8 = acc
    dminr = jnp.min(dmin8, axis=0, keepdims=True)

    svec = jnp.zeros((1, 128), jnp.float32)
    nvec = jnp.zeros((1, 128), jnp.float32)
    for q in range(BN // 128):
        dq = dminr[:, q * 128:(q + 1) * 128] + c2l[q:q + 1, :]
        dq = jnp.maximum(dq, 0.0)
        mask = dq < THRESHOLD
        svec = svec + jnp.where(mask, dq, 0.0)
        nvec = nvec + mask.astype(jnp.float32)

    s_ref[...] += svec
    n_ref[...] += nvec

    @pl.when(step == B * NBLK - 1)
    def _finish():
        s = jnp.sum(s_ref[...])
        mm = jnp.sum(n_ref[...])
        out_ref[0, 0] = jnp.where(mm > 0.0, s / (mm + 1e-6), 0.0)


@jax.jit
def kernel(completed, partial):
    ci = jnp.argsort(completed[..., 0], axis=1)
    c_s = jnp.take_along_axis(completed, ci[..., None], axis=1)
    pi = jnp.argsort(partial[..., 0], axis=1)
    p_s = jnp.take_along_axis(partial, pi[..., None], axis=1)

    pg = jnp.pad(p_s, ((0, 0), (0, 0), (0, 5)))
    at = jnp.transpose(-2.0 * c_s, (0, 2, 1))
    at = jnp.pad(at, ((0, 0), (0, 5), (0, 0)))
    c2 = jnp.sum(c_s * c_s, axis=-1)
    c2l = c2.reshape(B, N // 128, 128)

    csx = c_s[..., 0].reshape(B, NBLK, BN)
    cmin = csx[:, :, 0] - RAD
    cmax = csx[:, :, -1] + RAD
    psx = p_s[..., 0]
    lo = jax.vmap(lambda xs, v: jnp.searchsorted(xs, v, side='left'))(psx, cmin)
    hi = jax.vmap(lambda xs, v: jnp.searchsorted(xs, v, side='right'))(psx, cmax)
    loc = (lo // PCH).astype(jnp.int32).reshape(-1)
    hic = jnp.minimum((hi + PCH - 1) // PCH, NPCH).astype(jnp.int32).reshape(-1)
    hic = jnp.maximum(hic, loc)

    grid_spec = pltpu.PrefetchScalarGridSpec(
        num_scalar_prefetch=2,
        grid=(B, NBLK),
        in_specs=[
            pl.BlockSpec((1, M, 8), lambda b, i, loc, hic: (b, 0, 0)),
            pl.BlockSpec((1, 8, BN), lambda b, i, loc, hic: (b, 0, i)),
            pl.BlockSpec((1, BN // 128, 128), lambda b, i, loc, hic: (b, i, 0)),
        ],
        out_specs=pl.BlockSpec(memory_space=pltpu.SMEM),
        scratch_shapes=[
            pltpu.VMEM((1, 128), jnp.float32),
            pltpu.VMEM((1, 128), jnp.float32),
            pltpu.VMEM((M, 128), jnp.float32),
        ],
    )
    out = pl.pallas_call(
        _loss_kernel,
        grid_spec=grid_spec,
        out_shape=jax.ShapeDtypeStruct((1, 1), jnp.float32),
    )(loc, hic, pg, at, c2l)
    return WEIGHT * out[0, 0]


# speedup vs baseline: 2.5511x; 2.5511x over previous
import jax
import jax.numpy as jnp
from jax.experimental import pallas as pl
from jax.experimental.pallas import tpu as pltpu

THRESHOLD = 0.05
WEIGHT = 1.0

B = 8
N = 16384
M = 2048
BN = 8192
NBLK = N // BN
CH = 256
NCH = BN // CH


def _loss_kernel(pg_ref, at_ref, c2l_ref, out_ref, s_ref, n_ref, p2s_ref):
    b = pl.program_id(0)
    i = pl.program_id(1)
    step = b * NBLK + i

    @pl.when(step == 0)
    def _init():
        s_ref[...] = jnp.zeros_like(s_ref)
        n_ref[...] = jnp.zeros_like(n_ref)

    pg = pg_ref[0]
    at = at_ref[0]
    c2l = c2l_ref[0]

    @pl.when(i == 0)
    def _batch_setup():
        p2 = jnp.sum(pg * pg, axis=1, keepdims=True)
        p2s_ref[...] = jnp.broadcast_to(p2, (M, CH))

    p2s = p2s_ref[...]

    svec = jnp.zeros((1, CH), jnp.float32)
    nvec = jnp.zeros((1, CH), jnp.float32)
    for q in range(NCH):
        atc = at[:, q * CH:(q + 1) * CH]
        e = jnp.dot(pg, atc, preferred_element_type=jnp.float32)
        e = e + p2s
        rows = M
        while rows > 8:
            half = rows // 2
            e = jnp.minimum(e[:half], e[half:rows])
            rows = half
        dminc = jnp.min(e, axis=0, keepdims=True)
        dminc = jnp.maximum(dminc + c2l[q:q + 1, :], 0.0)
        mask = dminc < THRESHOLD
        svec = svec + jnp.where(mask, dminc, 0.0)
        nvec = nvec + mask.astype(jnp.float32)

    s_ref[...] += svec
    n_ref[...] += nvec

    @pl.when(step == B * NBLK - 1)
    def _finish():
        s = jnp.sum(s_ref[...])
        mm = jnp.sum(n_ref[...])
        out_ref[0, 0] = jnp.where(mm > 0.0, s / (mm + 1e-6), 0.0)


@jax.jit
def kernel(completed, partial):
    pg = jnp.pad(partial, ((0, 0), (0, 0), (0, 5)))
    at = jnp.transpose(-2.0 * completed, (0, 2, 1))
    at = jnp.pad(at, ((0, 0), (0, 5), (0, 0)))
    c2 = jnp.sum(completed * completed, axis=-1)
    c2l = c2.reshape(B, N // CH, CH)

    out = pl.pallas_call(
        _loss_kernel,
        grid=(B, NBLK),
        in_specs=[
            pl.BlockSpec((1, M, 8), lambda b, i: (b, 0, 0)),
            pl.BlockSpec((1, 8, BN), lambda b, i: (b, 0, i)),
            pl.BlockSpec((1, BN // CH, CH), lambda b, i: (b, i, 0)),
        ],
        out_specs=pl.BlockSpec(memory_space=pltpu.SMEM),
        out_shape=jax.ShapeDtypeStruct((1, 1), jnp.float32),
        scratch_shapes=[
            pltpu.VMEM((1, CH), jnp.float32),
            pltpu.VMEM((1, CH), jnp.float32),
            pltpu.VMEM((M, CH), jnp.float32),
        ],
    )(pg, at, c2l)
    return WEIGHT * out[0, 0]


# speedup vs baseline: 2.6445x; 1.0366x over previous
import jax
import jax.numpy as jnp
from jax.experimental import pallas as pl
from jax.experimental.pallas import tpu as pltpu

THRESHOLD = 0.05
WEIGHT = 1.0

B = 8
N = 16384
M = 2048
BN = 16384
NBLK = N // BN
CH = 256
NCH = BN // CH


def _loss_kernel(pg_ref, at_ref, c2l_ref, out_ref, s_ref, n_ref, p2s_ref):
    b = pl.program_id(0)
    i = pl.program_id(1)
    step = b * NBLK + i

    @pl.when(step == 0)
    def _init():
        s_ref[...] = jnp.zeros_like(s_ref)
        n_ref[...] = jnp.zeros_like(n_ref)

    pg = pg_ref[0]
    at = at_ref[0]
    c2l = c2l_ref[0]

    @pl.when(i == 0)
    def _batch_setup():
        p2 = jnp.sum(pg * pg, axis=1, keepdims=True)
        p2s_ref[...] = jnp.broadcast_to(p2, (M, CH))

    p2s = p2s_ref[...]

    svec = jnp.zeros((1, CH), jnp.float32)
    nvec = jnp.zeros((1, CH), jnp.float32)
    for q in range(NCH):
        atc = at[:, q * CH:(q + 1) * CH]
        e = jnp.dot(pg, atc, preferred_element_type=jnp.float32)
        e = e + p2s
        rows = M
        while rows > 8:
            half = rows // 2
            e = jnp.minimum(e[:half], e[half:rows])
            rows = half
        dminc = jnp.min(e, axis=0, keepdims=True)
        dminc = jnp.maximum(dminc + c2l[q:q + 1, :], 0.0)
        mask = dminc < THRESHOLD
        svec = svec + jnp.where(mask, dminc, 0.0)
        nvec = nvec + mask.astype(jnp.float32)

    s_ref[...] += svec
    n_ref[...] += nvec

    @pl.when(step == B * NBLK - 1)
    def _finish():
        s = jnp.sum(s_ref[...])
        mm = jnp.sum(n_ref[...])
        out_ref[0, 0] = jnp.where(mm > 0.0, s / (mm + 1e-6), 0.0)


@jax.jit
def kernel(completed, partial):
    pg = jnp.pad(partial, ((0, 0), (0, 0), (0, 5)))
    at = jnp.transpose(-2.0 * completed, (0, 2, 1))
    at = jnp.pad(at, ((0, 0), (0, 5), (0, 0)))
    c2 = jnp.sum(completed * completed, axis=-1)
    c2l = c2.reshape(B, N // CH, CH)

    out = pl.pallas_call(
        _loss_kernel,
        grid=(B, NBLK),
        in_specs=[
            pl.BlockSpec((1, M, 8), lambda b, i: (b, 0, 0)),
            pl.BlockSpec((1, 8, BN), lambda b, i: (b, 0, i)),
            pl.BlockSpec((1, BN // CH, CH), lambda b, i: (b, i, 0)),
        ],
        out_specs=pl.BlockSpec(memory_space=pltpu.SMEM),
        out_shape=jax.ShapeDtypeStruct((1, 1), jnp.float32),
        scratch_shapes=[
            pltpu.VMEM((1, CH), jnp.float32),
            pltpu.VMEM((1, CH), jnp.float32),
            pltpu.VMEM((M, CH), jnp.float32),
        ],
    )(pg, at, c2l)
    return WEIGHT * out[0, 0]
